# baseline (speedup 1.0000x reference)
import functools

import jax
import jax.numpy as jnp
from jax import lax
from jax.experimental import pallas as pl
from jax.experimental.pallas import tpu as pltpu
from jax.experimental.pallas import tpu_sc as plsc

NC, NS, L = 2, 16, 16
C = 128
N_SCALE = 12500
NSC_PAD = 12544
N_PTS = 100000
N_PTS_PAD = 102400
ROWS_PTS = 800
LOG2 = 0.6931471805599453
TOTAL = float(2 * 128 * 128 * 16)


def _mesh():
    return plsc.VectorSubcoreMesh(core_axis_name="c", subcore_axis_name="s")


def _fill2d(ref, rows, val):
    cols = ref.shape[1]

    def body(i, _):
        for k in range(cols // 16):
            ref[i, pl.ds(k * 16, 16)] = jnp.full((16,), val, ref.dtype)
        return 0

    lax.fori_loop(0, rows, body, 0)


@functools.partial(
    pl.kernel,
    out_type=(jax.ShapeDtypeStruct((NSC_PAD, C), jnp.float32),
              jax.ShapeDtypeStruct((NSC_PAD, C), jnp.float32)),
    mesh=_mesh(),
    scratch_types=(
        pltpu.VMEM_SHARED((NSC_PAD, C), jnp.float32),
        pltpu.VMEM((32,), jnp.int32),
        pltpu.VMEM((32,), jnp.int32),
        pltpu.VMEM((32, C), jnp.float32),
        pltpu.SemaphoreType.DMA,
    ),
)
def _sk4(lo_h, cl_h, cs_h, zeros_h, acc_o, cnt_o, spm, cl_v, cs_v, rows_v, sem):
    c = lax.axis_index("c")
    s = lax.axis_index("s")
    pltpu.sync_copy(zeros_h, spm.at[pl.ds(s * 784, 784), :])
    _fill2d(rows_v, 32, 1.0)
    plsc.subcore_barrier()

    base = s * 6400

    def body_sum(j, _):
        pltpu.sync_copy(cl_h.at[pl.ds(base + j * 32, 32)], cl_v)
        pltpu.sync_copy(cs_h.at[pl.ds(base + j * 32, 32)], cs_v)
        pltpu.async_copy(lo_h.at[cl_v], rows_v, sem).wait()
        pltpu.sync_copy(rows_v, spm.at[cs_v], add=True)
        return 0

    def body_cnt(j, _):
        pltpu.sync_copy(cs_h.at[pl.ds(base + j * 32, 32)], cs_v)
        pltpu.sync_copy(rows_v, spm.at[cs_v], add=True)
        return 0

    @pl.when(c == 0)
    def _():
        lax.fori_loop(0, 200, body_sum, 0)

    @pl.when(c == 1)
    def _():
        lax.fori_loop(0, 200, body_cnt, 0)

    plsc.subcore_barrier()

    @pl.when(c == 0)
    def _():
        pltpu.sync_copy(spm.at[pl.ds(s * 784, 784), :],
                        acc_o.at[pl.ds(s * 784, 784), :])

    @pl.when(c == 1)
    def _():
        pltpu.sync_copy(spm.at[pl.ds(s * 784, 784), :],
                        cnt_o.at[pl.ds(s * 784, 784), :])


@functools.partial(
    pl.kernel,
    out_type=jax.ShapeDtypeStruct((N_PTS_PAD, C), jnp.float32),
    mesh=_mesh(),
    scratch_types=(
        pltpu.VMEM((2, 128), jnp.int32),
        pltpu.VMEM((2, 128, C), jnp.float32),
        pltpu.SemaphoreType.DMA,
    ),
)
def _sk5(pfea_h, cs_h, out_h, cs_v, rows_v, sem):
    c = lax.axis_index("c")
    s = lax.axis_index("s")
    base_row = c * 400 + s * 25

    def body(j, _):
        pltpu.sync_copy(cs_h.at[base_row + j], cs_v.at[0])
        pltpu.async_copy(pfea_h.at[cs_v.at[0]], rows_v.at[0], sem).wait()
        pltpu.sync_copy(rows_v.at[0],
                        out_h.at[pl.ds((base_row + j) * 128, 128), :])
        return 0

    lax.fori_loop(0, 25, body, 0)


def _combine_body(a_ref, c_ref, o_ref):
    o_ref[...] = a_ref[...] / jnp.maximum(c_ref[:, 0:1], 1.0)


def _combine(acc, cnt):
    grid = (NSC_PAD // 896,)
    return pl.pallas_call(
        _combine_body,
        grid=grid,
        in_specs=[
            pl.BlockSpec((896, C), lambda i: (i, 0)),
            pl.BlockSpec((896, C), lambda i: (i, 0)),
        ],
        out_specs=pl.BlockSpec((896, C), lambda i: (i, 0)),
        out_shape=jax.ShapeDtypeStruct((NSC_PAD, C), jnp.float32),
    )(acc, cnt)


def _mm_lrelu_body(x_ref, w_ref, b_ref, o_ref):
    y = jnp.dot(x_ref[...], w_ref[...], preferred_element_type=jnp.float32) + b_ref[...]
    o_ref[...] = jnp.where(y > 0, y, 0.1 * y)


def _mm_lrelu(x, w, b, blk=2048):
    n, k = x.shape
    m = w.shape[1]
    grid = (pl.cdiv(n, blk),)
    return pl.pallas_call(
        _mm_lrelu_body,
        grid=grid,
        in_specs=[
            pl.BlockSpec((blk, k), lambda i: (i, 0)),
            pl.BlockSpec((k, m), lambda i: (0, 0)),
            pl.BlockSpec((1, m), lambda i: (0, 0)),
        ],
        out_specs=pl.BlockSpec((blk, m), lambda i: (i, 0)),
        out_shape=jax.ShapeDtypeStruct((n, m), jnp.float32),
    )(x, w, b.reshape(1, -1))


def kernel(features, partial_features, params, coors, coors_inv_last, coors_inv_scale):
    p = params

    def _bn(x):
        m = jnp.mean(x, axis=0)
        v = jnp.var(x, axis=0)
        return (x - m) / jnp.sqrt(v + 1e-5)

    def _lrelu(x):
        return jnp.where(x > 0, x, 0.1 * x)

    def _block(x, W1, b1, W2, b2):
        out = jax.nn.relu(_bn(x @ W1 + b1))
        out = _bn(out @ W2 + b2)
        return jax.nn.relu(out + x)

    v = _block(_block(features, p['v1_W1'], p['v1_b1'], p['v1_W2'], p['v1_b2']),
               p['v2_W1'], p['v2_b1'], p['v2_W2'], p['v2_b2'])
    vp = _block(_block(partial_features, p['v1_W1'], p['v1_b1'], p['v1_W2'], p['v1_b2']),
                p['v2_W1'], p['v2_b1'], p['v2_W2'], p['v2_b2'])
    logits = vp @ p['lg_W'] + p['lg_b']
    loss = (jnp.sum(jax.nn.softplus(-logits)) + (TOTAL - logits.shape[0]) * LOG2) / TOTAL

    feat = features + v
    n_max = feat.shape[0]
    key = (coors[:, 0] * (1 << 18) + (coors[:, 1] // 2) * (1 << 12)
           + (coors[:, 2] // 2) * (1 << 6) + (coors[:, 3] // 2))
    pres = jnp.zeros((1 << 19,), jnp.int32).at[key].set(1)
    ranks = jnp.cumsum(pres) - pres
    inv = ranks[key]

    cnt_seg = jnp.zeros((n_max, 1), jnp.float32).at[inv].add(1.0)
    sums = jnp.zeros((n_max, C), jnp.float32).at[inv].add(feat)
    down = sums / jnp.clip(cnt_seg, 1.0)
    seg_mask = (cnt_seg > 0).astype(jnp.float32)
    n_down_f = jnp.sum(seg_mask)

    def _bn_masked(x):
        m = jnp.sum(x * seg_mask, axis=0) / n_down_f
        v_ = jnp.sum(((x - m) ** 2) * seg_mask, axis=0) / n_down_f
        return (x - m) / jnp.sqrt(v_ + 1e-5)

    identity = _mm_lrelu(feat, p['pi_W'], p['pi_b'])
    pp = _lrelu(down @ p['pp_W1'] + p['pp_b1'])
    pp = _bn_masked(pp)
    pp = _lrelu(pp @ p['pp_W2'] + p['pp_b2'])
    pp = _bn_masked(pp)
    pp = _lrelu(pp @ p['pp_W3'] + p['pp_b3'])
    A = p['po_W1'][:C]
    B = p['po_W1'][C:]
    ident2 = _lrelu(identity @ A + pp[inv] @ B + p['po_b1'])
    lo50 = ident2 @ p['po_W2'] + p['po_b2']

    cl_pad = jnp.concatenate(
        [coors_inv_last, jnp.zeros((N_PTS_PAD - N_PTS,), jnp.int32)])
    cs_pad = jnp.concatenate(
        [coors_inv_scale, jnp.full((N_PTS_PAD - N_PTS,), N_SCALE, jnp.int32)])
    lo50_pad = jnp.concatenate([lo50, jnp.zeros((48, C), jnp.float32)], axis=0)
    zeros_h = jnp.zeros((784, C), jnp.float32)
    acc, cnt = _sk4(lo50_pad, cl_pad, cs_pad, zeros_h)
    p_fea = _combine(acc, cnt)
    out = _sk5(p_fea, cs_pad.reshape(ROWS_PTS, 128))
    return (out[:N_PTS], loss)


# speedup vs baseline: 1.1586x; 1.1586x over previous
import functools

import jax
import jax.numpy as jnp
from jax import lax
from jax.experimental import pallas as pl
from jax.experimental.pallas import tpu as pltpu
from jax.experimental.pallas import tpu_sc as plsc

NC, NS, L = 2, 16, 16
C = 128
N_SCALE = 12500
NSC_PAD = 12544
N_PTS = 100000
N_PTS_PAD = 102400
ROWS_PTS = 800
LOG2 = 0.6931471805599453
TOTAL = float(2 * 128 * 128 * 16)


def _mesh():
    return plsc.VectorSubcoreMesh(core_axis_name="c", subcore_axis_name="s")


def _fill2d(ref, rows, val):
    cols = ref.shape[1]

    def body(i, _):
        for k in range(cols // 16):
            ref[i, pl.ds(k * 16, 16)] = jnp.full((16,), val, ref.dtype)
        return 0

    lax.fori_loop(0, rows, body, 0)


@functools.partial(
    pl.kernel,
    out_type=(jax.ShapeDtypeStruct((NSC_PAD, C), jnp.float32),
              jax.ShapeDtypeStruct((NSC_PAD, C), jnp.float32)),
    mesh=_mesh(),
    scratch_types=(
        pltpu.VMEM_SHARED((NSC_PAD, C), jnp.float32),
        pltpu.VMEM((2, 64), jnp.int32),
        pltpu.VMEM((2, 64), jnp.int32),
        pltpu.VMEM((2, 64, C), jnp.float32),
        pltpu.SemaphoreType.DMA((2,)),
    ),
)
def _sk4(lo_h, cl_h, cs_h, zeros_h, acc_o, cnt_o, spm, cl_v, cs_v, rows_v, sems):
    c = lax.axis_index("c")
    s = lax.axis_index("s")
    pltpu.sync_copy(zeros_h, spm.at[pl.ds(s * 784, 784), :])
    _fill2d(rows_v.at[0], 64, 1.0)
    plsc.subcore_barrier()

    base = s * 6400
    NCHUNK = 100

    def start(j):
        b = j % 2
        pltpu.sync_copy(cl_h.at[pl.ds(base + j * 64, 64)], cl_v.at[b])
        pltpu.sync_copy(cs_h.at[pl.ds(base + j * 64, 64)], cs_v.at[b])
        pltpu.async_copy(lo_h.at[cl_v.at[b]], rows_v.at[b], sems.at[b])

    @pl.when(c == 0)
    def _():
        start(0)

        def body_sum(j, _):
            b = j % 2

            @pl.when(j + 1 < NCHUNK)
            def _():
                start(j + 1)

            pltpu.make_async_copy(lo_h.at[cl_v.at[b]], rows_v.at[b],
                                  sems.at[b]).wait()
            pltpu.sync_copy(rows_v.at[b], spm.at[cs_v.at[b]], add=True)
            return 0

        lax.fori_loop(0, NCHUNK, body_sum, 0)

    @pl.when(c == 1)
    def _():
        def body_cnt(j, _):
            pltpu.sync_copy(cs_h.at[pl.ds(base + j * 64, 64)], cs_v.at[0])
            pltpu.sync_copy(rows_v.at[0], spm.at[cs_v.at[0]], add=True)
            return 0

        lax.fori_loop(0, NCHUNK, body_cnt, 0)

    plsc.subcore_barrier()

    @pl.when(c == 0)
    def _():
        pltpu.sync_copy(spm.at[pl.ds(s * 784, 784), :],
                        acc_o.at[pl.ds(s * 784, 784), :])

    @pl.when(c == 1)
    def _():
        pltpu.sync_copy(spm.at[pl.ds(s * 784, 784), :],
                        cnt_o.at[pl.ds(s * 784, 784), :])


@functools.partial(
    pl.kernel,
    out_type=jax.ShapeDtypeStruct((N_PTS_PAD, C), jnp.float32),
    mesh=_mesh(),
    scratch_types=(
        pltpu.VMEM((2, 128), jnp.int32),
        pltpu.VMEM((2, 128, C), jnp.float32),
        pltpu.SemaphoreType.DMA((2,)),
    ),
)
def _sk5(pfea_h, cs_h, out_h, cs_v, rows_v, sems):
    c = lax.axis_index("c")
    s = lax.axis_index("s")
    base_row = c * 400 + s * 25

    def start(j):
        b = j % 2
        pltpu.sync_copy(cs_h.at[base_row + j], cs_v.at[b])
        pltpu.async_copy(pfea_h.at[cs_v.at[b]], rows_v.at[b], sems.at[b])

    start(0)

    def body(j, _):
        b = j % 2

        @pl.when(j + 1 < 25)
        def _():
            start(j + 1)

        pltpu.make_async_copy(pfea_h.at[cs_v.at[b]], rows_v.at[b],
                              sems.at[b]).wait()
        pltpu.sync_copy(rows_v.at[b],
                        out_h.at[pl.ds((base_row + j) * 128, 128), :])
        return 0

    lax.fori_loop(0, 25, body, 0)


def _combine_body(a_ref, c_ref, o_ref):
    o_ref[...] = a_ref[...] / jnp.maximum(c_ref[:, 0:1], 1.0)


def _combine(acc, cnt):
    grid = (NSC_PAD // 896,)
    return pl.pallas_call(
        _combine_body,
        grid=grid,
        in_specs=[
            pl.BlockSpec((896, C), lambda i: (i, 0)),
            pl.BlockSpec((896, C), lambda i: (i, 0)),
        ],
        out_specs=pl.BlockSpec((896, C), lambda i: (i, 0)),
        out_shape=jax.ShapeDtypeStruct((NSC_PAD, C), jnp.float32),
    )(acc, cnt)


def _mm_lrelu_body(x_ref, w_ref, b_ref, o_ref):
    y = jnp.dot(x_ref[...], w_ref[...], preferred_element_type=jnp.float32) + b_ref[...]
    o_ref[...] = jnp.where(y > 0, y, 0.1 * y)


def _mm_lrelu(x, w, b, blk=2048):
    n, k = x.shape
    m = w.shape[1]
    grid = (pl.cdiv(n, blk),)
    return pl.pallas_call(
        _mm_lrelu_body,
        grid=grid,
        in_specs=[
            pl.BlockSpec((blk, k), lambda i: (i, 0)),
            pl.BlockSpec((k, m), lambda i: (0, 0)),
            pl.BlockSpec((1, m), lambda i: (0, 0)),
        ],
        out_specs=pl.BlockSpec((blk, m), lambda i: (i, 0)),
        out_shape=jax.ShapeDtypeStruct((n, m), jnp.float32),
    )(x, w, b.reshape(1, -1))


def kernel(features, partial_features, params, coors, coors_inv_last, coors_inv_scale):
    p = params

    def _bn(x):
        m = jnp.mean(x, axis=0)
        v = jnp.var(x, axis=0)
        return (x - m) / jnp.sqrt(v + 1e-5)

    def _lrelu(x):
        return jnp.where(x > 0, x, 0.1 * x)

    def _block(x, W1, b1, W2, b2):
        out = jax.nn.relu(_bn(x @ W1 + b1))
        out = _bn(out @ W2 + b2)
        return jax.nn.relu(out + x)

    v = _block(_block(features, p['v1_W1'], p['v1_b1'], p['v1_W2'], p['v1_b2']),
               p['v2_W1'], p['v2_b1'], p['v2_W2'], p['v2_b2'])
    vp = _block(_block(partial_features, p['v1_W1'], p['v1_b1'], p['v1_W2'], p['v1_b2']),
                p['v2_W1'], p['v2_b1'], p['v2_W2'], p['v2_b2'])
    logits = vp @ p['lg_W'] + p['lg_b']
    loss = (jnp.sum(jax.nn.softplus(-logits)) + (TOTAL - logits.shape[0]) * LOG2) / TOTAL

    feat = features + v
    n_max = feat.shape[0]
    key = (coors[:, 0] * (1 << 18) + (coors[:, 1] // 2) * (1 << 12)
           + (coors[:, 2] // 2) * (1 << 6) + (coors[:, 3] // 2))
    pres = jnp.zeros((1 << 19,), jnp.int32).at[key].set(1)
    ranks = jnp.cumsum(pres) - pres
    inv = ranks[key]

    cnt_seg = jnp.zeros((n_max, 1), jnp.float32).at[inv].add(1.0)
    sums = jnp.zeros((n_max, C), jnp.float32).at[inv].add(feat)
    down = sums / jnp.clip(cnt_seg, 1.0)
    seg_mask = (cnt_seg > 0).astype(jnp.float32)
    n_down_f = jnp.sum(seg_mask)

    def _bn_masked(x):
        m = jnp.sum(x * seg_mask, axis=0) / n_down_f
        v_ = jnp.sum(((x - m) ** 2) * seg_mask, axis=0) / n_down_f
        return (x - m) / jnp.sqrt(v_ + 1e-5)

    identity = _mm_lrelu(feat, p['pi_W'], p['pi_b'])
    pp = _lrelu(down @ p['pp_W1'] + p['pp_b1'])
    pp = _bn_masked(pp)
    pp = _lrelu(pp @ p['pp_W2'] + p['pp_b2'])
    pp = _bn_masked(pp)
    pp = _lrelu(pp @ p['pp_W3'] + p['pp_b3'])
    A = p['po_W1'][:C]
    B = p['po_W1'][C:]
    ident2 = _lrelu(identity @ A + pp[inv] @ B + p['po_b1'])
    lo50 = ident2 @ p['po_W2'] + p['po_b2']

    cl_pad = jnp.concatenate(
        [coors_inv_last, jnp.zeros((N_PTS_PAD - N_PTS,), jnp.int32)])
    cs_pad = jnp.concatenate(
        [coors_inv_scale, jnp.full((N_PTS_PAD - N_PTS,), N_SCALE, jnp.int32)])
    lo50_pad = jnp.concatenate([lo50, jnp.zeros((48, C), jnp.float32)], axis=0)
    zeros_h = jnp.zeros((784, C), jnp.float32)
    acc, cnt = _sk4(lo50_pad, cl_pad, cs_pad, zeros_h)
    p_fea = _combine(acc, cnt)
    out = _sk5(p_fea, cs_pad.reshape(ROWS_PTS, 128))
    return (out[:N_PTS], loss)
